# Initial kernel scaffold; baseline (speedup 1.0000x reference)
import jax
import jax.numpy as jnp
from jax.experimental import pallas as pl


def kernel(x, perm):
    raise NotImplementedError("write your pallas kernel here")


# speedup vs baseline: 1.1249x; 1.1249x over previous
import functools

import jax
import jax.numpy as jnp
from jax import lax
from jax.experimental import pallas as pl
from jax.experimental.pallas import tpu as pltpu
from jax.experimental.pallas import tpu_sc as plsc

_B, _C, _H, _W = 4, 96, 384, 384
_P = 16
_HP = _H // _P
_WP = _W // _P
_L = _HP * _WP
_NPATCH = _B * _L
_SEGS = _C * _P
_TROWS = _B * _C * _H * _W // 16
_CSTRIDE = _H * _WP
_BSTRIDE = _C * _CSTRIDE

_NW = 32
_PPW = _NPATCH // _NW
_NDMA = _SEGS // 128


def _body(x_hbm, perm_hbm, out_hbm, perm_v, base_v, idx_v, data_v, gsem, ssem):
    cid = lax.axis_index("c")
    sid = lax.axis_index("s")
    wid = cid * 16 + sid
    g0 = wid * _PPW
    b = lax.div(g0, _L)

    pltpu.sync_copy(perm_hbm.at[pl.ds(g0, 80)], perm_v)

    for t in range(5):
        sv = perm_v[pl.ds(t * 16, 16)]
        wpv = jnp.full((16,), _WP, jnp.int32)
        base_v[pl.ds(t * 16, 16)] = (
            jnp.full((16,), b * _BSTRIDE, jnp.int32)
            + lax.div(sv, wpv) * (_P * _WP)
            + lax.rem(sv, wpv)
        )

    def patch_body(p, carry):
        pq = lax.div(p, 16)
        pr = lax.rem(p, 16)
        chunk = base_v[pl.ds(pq * 16, 16)]
        dnums = lax.GatherDimensionNumbers(
            offset_dims=(), collapsed_slice_dims=(0,), start_index_map=(0,)
        )
        basev = lax.gather(
            chunk,
            jnp.full((16, 1), pr, jnp.int32),
            dnums,
            (1,),
            mode=lax.GatherScatterMode.PROMISE_IN_BOUNDS,
        )

        def fill_c(c, _):
            di_off = lax.iota(jnp.int32, 16) * _WP
            idx_v[pl.ds(c * 16, 16)] = (
                basev + jnp.full((16,), c * _CSTRIDE, jnp.int32) + di_off
            )
            return 0

        lax.fori_loop(0, _C, fill_c, 0)

        copies = [
            pltpu.async_copy(
                x_hbm.at[idx_v.at[pl.ds(r * 128, 128)]],
                data_v.at[pl.ds(r * 128, 128)],
                gsem,
            )
            for r in range(_NDMA)
        ]
        for cp in copies:
            cp.wait()

        g = g0 + p
        pltpu.async_copy(
            data_v, out_hbm.at[pl.ds(g * _SEGS, _SEGS)], ssem
        ).wait()
        return carry

    lax.fori_loop(0, _PPW, patch_body, 0)


def kernel(x, perm):
    x2d = x.reshape(_TROWS, 16)
    perm_flat = jnp.pad(perm.reshape(_NPATCH), (0, 16))

    mesh = plsc.VectorSubcoreMesh(core_axis_name="c", subcore_axis_name="s")
    run = functools.partial(
        pl.kernel,
        mesh=mesh,
        out_type=jax.ShapeDtypeStruct((_TROWS, 16), jnp.float32),
        compiler_params=pltpu.CompilerParams(use_tc_tiling_on_sc=False),
        scratch_types=[
            pltpu.VMEM((80,), jnp.int32),
            pltpu.VMEM((80,), jnp.int32),
            pltpu.VMEM((_SEGS,), jnp.int32),
            pltpu.VMEM((_SEGS, 16), jnp.float32),
            pltpu.SemaphoreType.DMA,
            pltpu.SemaphoreType.DMA,
        ],
    )(_body)
    out2d = run(x2d, perm_flat)
    return out2d.reshape(_B, _L, _C, _P, _P)


# speedup vs baseline: 1.1597x; 1.0309x over previous
import functools

import jax
import jax.numpy as jnp
from jax import lax
from jax.experimental import pallas as pl
from jax.experimental.pallas import tpu as pltpu
from jax.experimental.pallas import tpu_sc as plsc

_B, _C, _H, _W = 4, 96, 384, 384
_P = 16
_HP = _H // _P
_WP = _W // _P
_L = _HP * _WP
_NPATCH = _B * _L
_SEGS = _C * _P
_TROWS = _B * _C * _H * _W // 16
_CSTRIDE = _H * _WP
_BSTRIDE = _C * _CSTRIDE

_NW = 32
_PPW = _NPATCH // _NW
_NBUF = 2


def _body(x_hbm, perm_hbm, out_hbm, perm_v, base_v, idx_v, data_v, gsems, ssems):
    cid = lax.axis_index("c")
    sid = lax.axis_index("s")
    wid = cid * 16 + sid
    g0 = wid * _PPW
    b = lax.div(g0, _L)

    pltpu.sync_copy(perm_hbm.at[pl.ds(g0, 80)], perm_v)

    for t in range(5):
        sv = perm_v[pl.ds(t * 16, 16)]
        wpv = jnp.full((16,), _WP, jnp.int32)
        base_v[pl.ds(t * 16, 16)] = (
            jnp.full((16,), b * _BSTRIDE, jnp.int32)
            + lax.div(sv, wpv) * (_P * _WP)
            + lax.rem(sv, wpv)
        )

    def fill_idx(p, s):
        pq = lax.div(p, 16)
        pr = lax.rem(p, 16)
        chunk = base_v[pl.ds(pq * 16, 16)]
        dnums = lax.GatherDimensionNumbers(
            offset_dims=(), collapsed_slice_dims=(0,), start_index_map=(0,)
        )
        basev = lax.gather(
            chunk,
            jnp.full((16, 1), pr, jnp.int32),
            dnums,
            (1,),
            mode=lax.GatherScatterMode.PROMISE_IN_BOUNDS,
        )

        def fill_c(cq, _):
            di_off = lax.iota(jnp.int32, 16) * _WP
            for u in range(4):
                c = cq * 4 + u
                idx_v[s, pl.ds(c * 16, 16)] = (
                    basev + jnp.full((16,), c * _CSTRIDE, jnp.int32) + di_off
                )
            return 0

        lax.fori_loop(0, _C // 4, fill_c, 0)

    def step_body(st, carry):
        copies = []
        for s in range(_NBUF):
            p = st * _NBUF + s

            @pl.when(st > 0)
            def _drain_scatter(s=s):
                pltpu.make_async_copy(
                    x_hbm.at[pl.ds(0, _SEGS)], data_v.at[s], ssems.at[s]
                ).wait()

            fill_idx(p, s)
            copies.append(
                pltpu.async_copy(
                    x_hbm.at[idx_v.at[s]], data_v.at[s], gsems.at[s]
                )
            )
        for s in range(_NBUF):
            p = st * _NBUF + s
            copies[s].wait()
            pltpu.async_copy(
                data_v.at[s],
                out_hbm.at[pl.ds((g0 + p) * _SEGS, _SEGS)],
                ssems.at[s],
            )
        return carry

    lax.fori_loop(0, _PPW // _NBUF, step_body, 0)

    for s in range(_NBUF):
        pltpu.make_async_copy(
            x_hbm.at[pl.ds(0, _SEGS)], data_v.at[s], ssems.at[s]
        ).wait()


def kernel(x, perm):
    x2d = x.reshape(_TROWS, 16)
    perm_flat = jnp.pad(perm.reshape(_NPATCH), (0, 16))

    mesh = plsc.VectorSubcoreMesh(core_axis_name="c", subcore_axis_name="s")
    run = functools.partial(
        pl.kernel,
        mesh=mesh,
        out_type=jax.ShapeDtypeStruct((_TROWS, 16), jnp.float32),
        compiler_params=pltpu.CompilerParams(use_tc_tiling_on_sc=False),
        scratch_types=[
            pltpu.VMEM((80,), jnp.int32),
            pltpu.VMEM((80,), jnp.int32),
            pltpu.VMEM((_NBUF, _SEGS), jnp.int32),
            pltpu.VMEM((_NBUF, _SEGS, 16), jnp.float32),
            pltpu.SemaphoreType.DMA((_NBUF,)),
            pltpu.SemaphoreType.DMA((_NBUF,)),
        ],
    )(_body)
    out2d = run(x2d, perm_flat)
    return out2d.reshape(_B, _L, _C, _P, _P)


# speedup vs baseline: 1.2718x; 1.0967x over previous
import functools

import jax
import jax.numpy as jnp
from jax import lax
from jax.experimental import pallas as pl
from jax.experimental.pallas import tpu as pltpu
from jax.experimental.pallas import tpu_sc as plsc

_B, _C, _H, _W = 4, 96, 384, 384
_P = 16
_HP = _H // _P
_WP = _W // _P
_L = _HP * _WP
_NPATCH = _B * _L
_SEGS = _C * _P
_TROWS = _B * _C * _H * _W // 16
_CSTRIDE = _H * _WP
_BSTRIDE = _C * _CSTRIDE

_NW = 32
_PPW = _NPATCH // _NW
_NBUF = 2


def _body(x_hbm, perm_hbm, out_hbm, perm_v, base_v, idx_v, data_v, gsems, ssems):
    cid = lax.axis_index("c")
    sid = lax.axis_index("s")
    wid = cid * 16 + sid
    g0 = wid * _PPW
    b = lax.div(g0, _L)

    pltpu.sync_copy(perm_hbm.at[pl.ds(g0, 80)], perm_v)

    for t in range(5):
        sv = perm_v[pl.ds(t * 16, 16)]
        wpv = jnp.full((16,), _WP, jnp.int32)
        e8 = jnp.full((16,), 8, jnp.int32)
        jv = lax.rem(sv, wpv)
        base_v[pl.ds(t * 16, 16)] = (
            jnp.full((16,), b * _BSTRIDE, jnp.int32)
            + lax.div(sv, wpv) * (_P * _WP)
            + lax.div(jv, e8) * 64
            + lax.rem(jv, e8)
        )

    def fill_idx(p, s):
        pq = lax.div(p, 16)
        pr = lax.rem(p, 16)
        chunk = base_v[pl.ds(pq * 16, 16)]
        dnums = lax.GatherDimensionNumbers(
            offset_dims=(), collapsed_slice_dims=(0,), start_index_map=(0,)
        )
        basev = lax.gather(
            chunk,
            jnp.full((16, 1), pr, jnp.int32),
            dnums,
            (1,),
            mode=lax.GatherScatterMode.PROMISE_IN_BOUNDS,
        )

        def fill_c(cq, _):
            lane = lax.iota(jnp.int32, 16)
            e8 = jnp.full((16,), 8, jnp.int32)
            di_off = lax.div(lane, e8) * 192 + lax.rem(lane, e8) * 8
            for u in range(4):
                c = cq * 4 + u
                idx_v[s, pl.ds(c * 16, 16)] = (
                    basev + jnp.full((16,), c * _CSTRIDE, jnp.int32) + di_off
                )
            return 0

        lax.fori_loop(0, _C // 4, fill_c, 0)

    def step_body(st, carry):
        copies = []
        for s in range(_NBUF):
            p = st * _NBUF + s

            @pl.when(st > 0)
            def _drain_scatter(s=s):
                pltpu.make_async_copy(
                    x_hbm.at[pl.ds(0, _SEGS)], data_v.at[s], ssems.at[s]
                ).wait()

            fill_idx(p, s)
            copies.append(
                pltpu.async_copy(
                    x_hbm.at[idx_v.at[s]], data_v.at[s], gsems.at[s]
                )
            )
        for s in range(_NBUF):
            p = st * _NBUF + s
            copies[s].wait()
            pltpu.async_copy(
                data_v.at[s],
                out_hbm.at[pl.ds((g0 + p) * _SEGS, _SEGS)],
                ssems.at[s],
            )
        return carry

    lax.fori_loop(0, _PPW // _NBUF, step_body, 0)

    for s in range(_NBUF):
        pltpu.make_async_copy(
            x_hbm.at[pl.ds(0, _SEGS)], data_v.at[s], ssems.at[s]
        ).wait()


def kernel(x, perm):
    x2d = (
        x.reshape(_B, _C, _H // 8, 8, _W // 128, 128)
        .transpose(0, 1, 2, 4, 3, 5)
        .reshape(_TROWS, 16)
    )
    perm_flat = jnp.pad(perm.reshape(_NPATCH), (0, 16))

    mesh = plsc.VectorSubcoreMesh(core_axis_name="c", subcore_axis_name="s")
    run = functools.partial(
        pl.kernel,
        mesh=mesh,
        out_type=jax.ShapeDtypeStruct((_TROWS, 16), jnp.float32),
        compiler_params=pltpu.CompilerParams(use_tc_tiling_on_sc=False),
        scratch_types=[
            pltpu.VMEM((80,), jnp.int32),
            pltpu.VMEM((80,), jnp.int32),
            pltpu.VMEM((_NBUF, _SEGS), jnp.int32),
            pltpu.VMEM((_NBUF, _SEGS, 16), jnp.float32),
            pltpu.SemaphoreType.DMA((_NBUF,)),
            pltpu.SemaphoreType.DMA((_NBUF,)),
        ],
    )(_body)
    out2d = run(x2d, perm_flat)
    return out2d.reshape(_B, _L, _C, _P, _P)
